# baseline (speedup 1.0000x reference)
import functools

import jax
import jax.numpy as jnp
from jax import lax
from jax.experimental import pallas as pl
from jax.experimental.pallas import tpu as pltpu
from jax.experimental.pallas import tpu_sc as plsc

VOCAB = 100000
D = 128
SEQ = 200
BATCH = 1024
ROWS = BATCH * SEQ

NC = 2
NS = 16
NW = NC * NS
ROWS_PER_W = ROWS // NW
CHUNK = 64
NCHUNK = ROWS_PER_W // CHUNK
NBUF = 10
AHEAD = 6
NGROUP = NCHUNK // NBUF


@functools.partial(
    pl.kernel,
    out_type=jax.ShapeDtypeStruct((ROWS, D), jnp.float32),
    mesh=plsc.VectorSubcoreMesh(core_axis_name="c", subcore_axis_name="s"),
    scratch_types=(
        [pltpu.VMEM((NCHUNK, CHUNK), jnp.int32),
         pltpu.VMEM((SEQ, D), jnp.float32)]
        + [pltpu.VMEM((CHUNK, D), jnp.float32)] * NBUF
        + [pltpu.SemaphoreType.DMA] * (2 * NBUF)
    ),
)
def _emb_body(x_hbm, tok_hbm, pos_hbm, out_hbm, idx_v, pos_v, *ring):
    bufs = ring[:NBUF]
    gsem = ring[NBUF:2 * NBUF]
    ssem = ring[2 * NBUF:]

    wid = lax.axis_index("s") * NC + lax.axis_index("c")
    base_chunk = wid * NCHUNK

    icopy = pltpu.async_copy(x_hbm.at[wid], idx_v, gsem[0])
    pcopy = pltpu.async_copy(pos_hbm, pos_v, ssem[0])
    icopy.wait()

    def out_slice(c):
        return out_hbm.at[pl.ds((base_chunk + c) * CHUNK, CHUNK)]

    def start_gather(c, b):
        pltpu.async_copy(tok_hbm.at[idx_v.at[c]], bufs[b], gsem[b])

    def wait_gather(c, b):
        pltpu.make_async_copy(tok_hbm.at[idx_v.at[c]], bufs[b], gsem[b]).wait()

    def start_store(c, b):
        pltpu.async_copy(bufs[b], out_slice(c), ssem[b])

    def wait_store(c, b):
        pltpu.make_async_copy(bufs[b], out_slice(c), ssem[b]).wait()

    for b in range(AHEAD):
        start_gather(b, b)
    pcopy.wait()

    def group(g, carry):
        for b in range(NBUF):
            c = g * NBUF + b
            wait_gather(c, b)
            p0 = lax.rem(c * CHUNK, SEQ)
            n1 = jnp.minimum(CHUNK, SEQ - p0)

            @plsc.parallel_loop(0, n1, unroll=4)
            def _(r):
                for k in range(D // 16):
                    sl = pl.ds(k * 16, 16)
                    plsc.addupdate(bufs[b].at[r, sl], pos_v[p0 + r, sl])

            @plsc.parallel_loop(n1, CHUNK, unroll=4)
            def _(r):
                for k in range(D // 16):
                    sl = pl.ds(k * 16, 16)
                    plsc.addupdate(bufs[b].at[r, sl], pos_v[r - n1, sl])

            start_store(c, b)

            nb = (b + AHEAD) % NBUF
            nc = c + AHEAD

            @pl.when(nc < NCHUNK)
            def _():
                @pl.when(c >= NBUF - AHEAD)
                def _():
                    wait_store(nc - NBUF, nb)
                start_gather(nc, nb)

        return carry

    lax.fori_loop(0, NGROUP, group, 0)

    for b in range(NBUF):
        wait_store(NCHUNK - NBUF + b, b)


def kernel(x, token_table, position_table):
    x2 = x.reshape(NW, NCHUNK, CHUNK).astype(jnp.int32)
    out = _emb_body(x2, token_table, position_table)
    return out.reshape(BATCH, SEQ, D)


# speedup vs baseline: 1.0664x; 1.0664x over previous
import functools

import jax
import jax.numpy as jnp
from jax import lax
from jax.experimental import pallas as pl
from jax.experimental.pallas import tpu as pltpu
from jax.experimental.pallas import tpu_sc as plsc

VOCAB = 100000
D = 128
SEQ = 200
BATCH = 1024
ROWS = BATCH * SEQ

NC = 2
NS = 16
NW = NC * NS
ROWS_PER_W = ROWS // NW
CHUNK = 128
NCHUNK = ROWS_PER_W // CHUNK
NBUF = 5
AHEAD = 3
NGROUP = NCHUNK // NBUF


@functools.partial(
    pl.kernel,
    out_type=jax.ShapeDtypeStruct((ROWS, D), jnp.float32),
    mesh=plsc.VectorSubcoreMesh(core_axis_name="c", subcore_axis_name="s"),
    scratch_types=(
        [pltpu.VMEM((NCHUNK, CHUNK), jnp.int32),
         pltpu.VMEM((SEQ, D), jnp.float32)]
        + [pltpu.VMEM((CHUNK, D), jnp.float32)] * NBUF
        + [pltpu.SemaphoreType.DMA] * (2 * NBUF)
    ),
)
def _emb_body(x_hbm, tok_hbm, pos_hbm, out_hbm, idx_v, pos_v, *ring):
    bufs = ring[:NBUF]
    gsem = ring[NBUF:2 * NBUF]
    ssem = ring[2 * NBUF:]

    wid = lax.axis_index("s") * NC + lax.axis_index("c")
    base_chunk = wid * NCHUNK

    icopy = pltpu.async_copy(x_hbm.at[wid], idx_v, gsem[0])
    pcopy = pltpu.async_copy(pos_hbm, pos_v, ssem[0])
    icopy.wait()

    def out_slice(c):
        return out_hbm.at[pl.ds((base_chunk + c) * CHUNK, CHUNK)]

    def start_gather(c, b):
        pltpu.async_copy(tok_hbm.at[idx_v.at[c]], bufs[b], gsem[b])

    def wait_gather(c, b):
        pltpu.make_async_copy(tok_hbm.at[idx_v.at[c]], bufs[b], gsem[b]).wait()

    def start_store(c, b):
        pltpu.async_copy(bufs[b], out_slice(c), ssem[b])

    def wait_store(c, b):
        pltpu.make_async_copy(bufs[b], out_slice(c), ssem[b]).wait()

    for b in range(AHEAD):
        start_gather(b, b)
    pcopy.wait()

    def group(g, carry):
        for b in range(NBUF):
            c = g * NBUF + b
            wait_gather(c, b)
            p0 = lax.rem(c * CHUNK, SEQ)
            n1 = jnp.minimum(CHUNK, SEQ - p0)

            @plsc.parallel_loop(0, n1, unroll=4)
            def _(r):
                for k in range(D // 16):
                    sl = pl.ds(k * 16, 16)
                    plsc.addupdate(bufs[b].at[r, sl], pos_v[p0 + r, sl])

            @plsc.parallel_loop(n1, CHUNK, unroll=4)
            def _(r):
                for k in range(D // 16):
                    sl = pl.ds(k * 16, 16)
                    plsc.addupdate(bufs[b].at[r, sl], pos_v[r - n1, sl])

            start_store(c, b)

            nb = (b + AHEAD) % NBUF
            nc = c + AHEAD

            @pl.when(nc < NCHUNK)
            def _():
                @pl.when(c >= NBUF - AHEAD)
                def _():
                    wait_store(nc - NBUF, nb)
                start_gather(nc, nb)

        return carry

    lax.fori_loop(0, NGROUP, group, 0)

    for b in range(NBUF):
        wait_store(NCHUNK - NBUF + b, b)


def kernel(x, token_table, position_table):
    x2 = x.reshape(NW, NCHUNK, CHUNK).astype(jnp.int32)
    out = _emb_body(x2, token_table, position_table)
    return out.reshape(BATCH, SEQ, D)
